# baseline (speedup 1.0000x reference)
import functools

import jax
import jax.numpy as jnp
from jax.experimental import pallas as pl
from jax.experimental.pallas import tpu as pltpu

BN_EPS = 1e-5
LEAKY_SLOPE = 0.2
_DOT_DT = jnp.bfloat16
_MID_DT = jnp.bfloat16


def _lrelu(z):
    return jnp.maximum(z, LEAKY_SLOPE * z)


def _dot(w, a):
    return jnp.dot(w, a.astype(_DOT_DT), preferred_element_type=jnp.float32)


def _mega_kernel(x_ref,
                 w1_0, g1_0, b1_0, w2_0, g2_0, b2_0,
                 w1_1, g1_1, b1_1, w2_1, g2_1, b2_1,
                 w1_2, g1_2, b1_2, w2_2, g2_2, b2_2,
                 o_ref,
                 buf_a, buf_b, sum1, ssq1, sum2, ssq2,
                 w1f, t1s, w2f, t2s,
                 *, bn, m_real):
    p = pl.program_id(0)
    j = pl.program_id(1)

    def fold(s_ref, q_ref, g_ref, b_ref):
        mean = s_ref[...] / m_real
        var = jnp.maximum(q_ref[...] / m_real - mean * mean, 0.0)
        scale = g_ref[...] * jax.lax.rsqrt(var + BN_EPS)
        shift = b_ref[...] - mean * scale
        return scale, shift

    def acc1(y):
        sum1[...] += jnp.sum(y, axis=1, keepdims=True)
        ssq1[...] += jnp.sum(y * y, axis=1, keepdims=True)

    def acc2(y):
        sum2[...] += jnp.sum(y, axis=1, keepdims=True)
        ssq2[...] += jnp.sum(y * y, axis=1, keepdims=True)

    @pl.when(p == 0)
    def _():
        @pl.when(j == 0)
        def _():
            sum1[...] = jnp.zeros_like(sum1)
            ssq1[...] = jnp.zeros_like(ssq1)

        w1b = w1_0[...].astype(_DOT_DT)
        ch = x_ref.shape[1]
        m = x_ref.shape[2] * x_ref.shape[3]
        for i in range(bn):
            xb = x_ref[i].reshape(ch, m).astype(_MID_DT)
            buf_a[j, i] = xb
            acc1(_dot(w1b, xb))

    def stats2_phase(ph, src, w1_b, g1_b, b1_b, w2_b):
        @pl.when(p == ph)
        def _():
            @pl.when(j == 0)
            def _():
                s1, t1v = fold(sum1, ssq1, g1_b, b1_b)
                w1f[...] = (w1_b[...] * s1).astype(_DOT_DT)
                t1s[...] = t1v
                sum2[...] = jnp.zeros_like(sum2)
                ssq2[...] = jnp.zeros_like(ssq2)

            w1fv = w1f[...]
            w2b = w2_b[...].astype(_DOT_DT)
            for i in range(bn):
                z1 = _lrelu(_dot(w1fv, src[j, i]) + t1s[...])
                acc2(_dot(w2b, z1))

    def ff_phase(ph, src, dst, w2_b, g2_b, b2_b, wn):
        @pl.when(p == ph)
        def _():
            @pl.when(j == 0)
            def _():
                s2, t2v = fold(sum2, ssq2, g2_b, b2_b)
                w2f[...] = (w2_b[...] * s2).astype(_DOT_DT)
                t2s[...] = t2v
                sum1[...] = jnp.zeros_like(sum1)
                ssq1[...] = jnp.zeros_like(ssq1)

            w1fv = w1f[...]
            w2fv = w2f[...]
            wnb = wn[...].astype(_DOT_DT)
            for i in range(bn):
                z1 = _lrelu(_dot(w1fv, src[j, i]) + t1s[...])
                z2b = _lrelu(_dot(w2fv, z1) + t2s[...]).astype(_MID_DT)
                dst[j, i] = z2b
                acc1(_dot(wnb, z2b))

    stats2_phase(1, buf_a, w1_0, g1_0, b1_0, w2_0)
    ff_phase(2, buf_a, buf_b, w2_0, g2_0, b2_0, w1_1)
    stats2_phase(3, buf_b, w1_1, g1_1, b1_1, w2_1)
    ff_phase(4, buf_b, buf_a, w2_1, g2_1, b2_1, w1_2)
    stats2_phase(5, buf_a, w1_2, g1_2, b1_2, w2_2)

    @pl.when(p == 6)
    def _():
        @pl.when(j == 0)
        def _():
            s2, t2v = fold(sum2, ssq2, g2_2, b2_2)
            w2f[...] = (w2_2[...] * s2).astype(_DOT_DT)
            t2s[...] = t2v

        w1fv = w1f[...]
        w2fv = w2f[...]
        ch, hh, ww = o_ref.shape[1], o_ref.shape[2], o_ref.shape[3]
        for i in range(bn):
            z1 = _lrelu(_dot(w1fv, buf_a[j, i]) + t1s[...])
            z2 = _lrelu(_dot(w2fv, z1) + t2s[...])
            o_ref[i] = z2.reshape(ch, hh, ww)


def kernel(x,
           w1_0, g1_0, b1_0, w2_0, g2_0, b2_0,
           w1_1, g1_1, b1_1, w2_1, g2_1, b2_1,
           w1_2, g1_2, b1_2, w2_2, g2_2, b2_2):
    n, c_in, h, w = x.shape
    hw = h * w
    m_real = n * hw
    c1 = w1_0.shape[0]
    c2 = w2_0.shape[0]

    bn = 2
    steps = -(-n // bn)
    grid = (7, steps)

    cp = pltpu.CompilerParams(
        dimension_semantics=("arbitrary", "arbitrary"),
        vmem_limit_bytes=56 * 1024 * 1024)

    params = [w1_0, g1_0, b1_0, w2_0, g2_0, b2_0,
              w1_1, g1_1, b1_1, w2_1, g2_1, b2_1,
              w1_2, g1_2, b1_2, w2_2, g2_2, b2_2]

    def full_spec(shape):
        nd = len(shape)
        return pl.BlockSpec(tuple(shape), lambda p, j: (0,) * nd)

    last = steps - 1
    x_spec = pl.BlockSpec(
        (bn, c_in, h, w),
        lambda p, j: (jnp.where(p == 0, j, last), 0, 0, 0))
    o_spec = pl.BlockSpec(
        (bn, c2, h, w),
        lambda p, j: (jnp.where(p == 6, j, 0), 0, 0, 0))

    out = pl.pallas_call(
        functools.partial(_mega_kernel, bn=bn, m_real=m_real),
        grid=grid,
        in_specs=[x_spec] + [full_spec(a.shape) for a in params],
        out_specs=o_spec,
        out_shape=jax.ShapeDtypeStruct((n, c2, h, w), jnp.float32),
        scratch_shapes=[
            pltpu.VMEM((steps, bn, c_in, hw), _MID_DT),
            pltpu.VMEM((steps, bn, c2, hw), _MID_DT),
            pltpu.VMEM((c1, 1), jnp.float32),
            pltpu.VMEM((c1, 1), jnp.float32),
            pltpu.VMEM((c2, 1), jnp.float32),
            pltpu.VMEM((c2, 1), jnp.float32),
            pltpu.VMEM((c1, c_in), _DOT_DT),
            pltpu.VMEM((c1, 1), jnp.float32),
            pltpu.VMEM((c2, c1), _DOT_DT),
            pltpu.VMEM((c2, 1), jnp.float32),
        ],
        compiler_params=cp,
    )(x, *params)
    return out


# speedup vs baseline: 1.0771x; 1.0771x over previous
import functools

import jax
import jax.numpy as jnp
from jax.experimental import pallas as pl
from jax.experimental.pallas import tpu as pltpu

BN_EPS = 1e-5
LEAKY_SLOPE = 0.2
_DOT_DT = jnp.bfloat16
_MID_DT = jnp.bfloat16


def _lrelu(z):
    return jnp.maximum(z, LEAKY_SLOPE * z)


def _dot(w, a):
    return jnp.dot(w, a.astype(_DOT_DT), preferred_element_type=jnp.float32)


def _fold(s_ref, q_ref, g_ref, b_ref, m_real):
    mean = s_ref[...] / m_real
    var = jnp.maximum(q_ref[...] / m_real - mean * mean, 0.0)
    scale = g_ref[...] * jax.lax.rsqrt(var + BN_EPS)
    shift = b_ref[...] - mean * scale
    return scale, shift


def _kernel_a(x_ref, w1_0, g1_0, b1_0, w2_0, g2_0, b2_0, w1_1,
              z0_ref, sum1o, ssq1o,
              buf_a, sum1, ssq1, sum2, ssq2, w1f, t1s, w2f, t2s,
              *, bn, steps, m_real):
    p = pl.program_id(0)
    j = pl.program_id(1)

    def acc(y, s_ref, q_ref):
        s_ref[...] += jnp.sum(y, axis=1, keepdims=True)
        q_ref[...] += jnp.sum(y * y, axis=1, keepdims=True)

    @pl.when(p == 0)
    def _():
        @pl.when(j == 0)
        def _():
            sum1[...] = jnp.zeros_like(sum1)
            ssq1[...] = jnp.zeros_like(ssq1)

        w1b = w1_0[...].astype(_DOT_DT)
        ch = x_ref.shape[1]
        m = x_ref.shape[2] * x_ref.shape[3]
        for i in range(bn):
            xb = x_ref[i].reshape(ch, m).astype(_MID_DT)
            buf_a[j, i] = xb
            acc(_dot(w1b, xb), sum1, ssq1)

    @pl.when(p == 1)
    def _():
        @pl.when(j == 0)
        def _():
            s1, t1v = _fold(sum1, ssq1, g1_0, b1_0, m_real)
            w1f[...] = (w1_0[...] * s1).astype(_DOT_DT)
            t1s[...] = t1v
            sum2[...] = jnp.zeros_like(sum2)
            ssq2[...] = jnp.zeros_like(ssq2)

        w1fv = w1f[...]
        w2b = w2_0[...].astype(_DOT_DT)
        for i in range(bn):
            z1 = _lrelu(_dot(w1fv, buf_a[j, i]) + t1s[...])
            acc(_dot(w2b, z1), sum2, ssq2)

    @pl.when(p == 2)
    def _():
        @pl.when(j == 0)
        def _():
            s2, t2v = _fold(sum2, ssq2, g2_0, b2_0, m_real)
            w2f[...] = (w2_0[...] * s2).astype(_DOT_DT)
            t2s[...] = t2v
            sum1[...] = jnp.zeros_like(sum1)
            ssq1[...] = jnp.zeros_like(ssq1)

        w1fv = w1f[...]
        w2fv = w2f[...]
        wnb = w1_1[...].astype(_DOT_DT)
        for i in range(bn):
            z1 = _lrelu(_dot(w1fv, buf_a[j, i]) + t1s[...])
            z2b = _lrelu(_dot(w2fv, z1) + t2s[...]).astype(_MID_DT)
            z0_ref[i] = z2b
            acc(_dot(wnb, z2b), sum1, ssq1)

        @pl.when(j == steps - 1)
        def _():
            sum1o[...] = sum1[...]
            ssq1o[...] = ssq1[...]


def _kernel_b(z0_ref, sum1i, ssq1i,
              w1_1, g1_1, b1_1, w2_1, g2_1, b2_1,
              w1_2, g1_2, b1_2, w2_2, g2_2, b2_2,
              o_ref,
              buf_a, buf_b, sum1, ssq1, sum2, ssq2, w1f, t1s, w2f, t2s,
              *, bn, m_real):
    p = pl.program_id(0)
    j = pl.program_id(1)

    def acc(y, s_ref, q_ref):
        s_ref[...] += jnp.sum(y, axis=1, keepdims=True)
        q_ref[...] += jnp.sum(y * y, axis=1, keepdims=True)

    @pl.when(p == 0)
    def _():
        @pl.when(j == 0)
        def _():
            s1, t1v = _fold(sum1i, ssq1i, g1_1, b1_1, m_real)
            w1f[...] = (w1_1[...] * s1).astype(_DOT_DT)
            t1s[...] = t1v
            sum2[...] = jnp.zeros_like(sum2)
            ssq2[...] = jnp.zeros_like(ssq2)

        w1fv = w1f[...]
        w2b = w2_1[...].astype(_DOT_DT)
        for i in range(bn):
            a = z0_ref[i]
            buf_a[j, i] = a
            z1 = _lrelu(_dot(w1fv, a) + t1s[...])
            acc(_dot(w2b, z1), sum2, ssq2)

    @pl.when(p == 1)
    def _():
        @pl.when(j == 0)
        def _():
            s2, t2v = _fold(sum2, ssq2, g2_1, b2_1, m_real)
            w2f[...] = (w2_1[...] * s2).astype(_DOT_DT)
            t2s[...] = t2v
            sum1[...] = jnp.zeros_like(sum1)
            ssq1[...] = jnp.zeros_like(ssq1)

        w1fv = w1f[...]
        w2fv = w2f[...]
        wnb = w1_2[...].astype(_DOT_DT)
        for i in range(bn):
            z1 = _lrelu(_dot(w1fv, buf_a[j, i]) + t1s[...])
            z2b = _lrelu(_dot(w2fv, z1) + t2s[...]).astype(_MID_DT)
            buf_b[j, i] = z2b
            acc(_dot(wnb, z2b), sum1, ssq1)

    @pl.when(p == 2)
    def _():
        @pl.when(j == 0)
        def _():
            s1, t1v = _fold(sum1, ssq1, g1_2, b1_2, m_real)
            w1f[...] = (w1_2[...] * s1).astype(_DOT_DT)
            t1s[...] = t1v
            sum2[...] = jnp.zeros_like(sum2)
            ssq2[...] = jnp.zeros_like(ssq2)

        w1fv = w1f[...]
        w2b = w2_2[...].astype(_DOT_DT)
        for i in range(bn):
            z1 = _lrelu(_dot(w1fv, buf_b[j, i]) + t1s[...])
            acc(_dot(w2b, z1), sum2, ssq2)

    @pl.when(p == 3)
    def _():
        @pl.when(j == 0)
        def _():
            s2, t2v = _fold(sum2, ssq2, g2_2, b2_2, m_real)
            w2f[...] = (w2_2[...] * s2).astype(_DOT_DT)
            t2s[...] = t2v

        w1fv = w1f[...]
        w2fv = w2f[...]
        ch, hh, ww = o_ref.shape[1], o_ref.shape[2], o_ref.shape[3]
        for i in range(bn):
            z1 = _lrelu(_dot(w1fv, buf_b[j, i]) + t1s[...])
            z2 = _lrelu(_dot(w2fv, z1) + t2s[...])
            o_ref[i] = z2.reshape(ch, hh, ww)


def kernel(x,
           w1_0, g1_0, b1_0, w2_0, g2_0, b2_0,
           w1_1, g1_1, b1_1, w2_1, g2_1, b2_1,
           w1_2, g1_2, b1_2, w2_2, g2_2, b2_2):
    n, c_in, h, w = x.shape
    hw = h * w
    m_real = n * hw
    c1 = w1_0.shape[0]
    c2 = w2_0.shape[0]

    bn = 4
    steps = -(-n // bn)

    cp = pltpu.CompilerParams(
        dimension_semantics=("arbitrary", "arbitrary"),
        vmem_limit_bytes=56 * 1024 * 1024)

    def full_spec(shape):
        nd = len(shape)
        return pl.BlockSpec(tuple(shape), lambda p, j: (0,) * nd)

    def acc_spec(ch):
        return pl.BlockSpec((ch, 1), lambda p, j: (0, 0))

    last = steps - 1
    x_spec = pl.BlockSpec(
        (bn, c_in, h, w),
        lambda p, j: (jnp.where(p == 0, j, last), 0, 0, 0))
    z0_out_spec = pl.BlockSpec(
        (bn, c2, hw), lambda p, j: (jnp.where(p == 2, j, 0), 0, 0))
    z0_in_spec = pl.BlockSpec(
        (bn, c2, hw), lambda p, j: (jnp.where(p == 0, j, last), 0, 0))
    o_spec = pl.BlockSpec(
        (bn, c2, h, w),
        lambda p, j: (jnp.where(p == 3, j, 0), 0, 0, 0))

    params_a = [w1_0, g1_0, b1_0, w2_0, g2_0, b2_0, w1_1]
    z0, sum1, ssq1 = pl.pallas_call(
        functools.partial(_kernel_a, bn=bn, steps=steps, m_real=m_real),
        grid=(3, steps),
        in_specs=[x_spec] + [full_spec(a.shape) for a in params_a],
        out_specs=(z0_out_spec, acc_spec(c1), acc_spec(c1)),
        out_shape=(jax.ShapeDtypeStruct((n, c2, hw), _MID_DT),
                   jax.ShapeDtypeStruct((c1, 1), jnp.float32),
                   jax.ShapeDtypeStruct((c1, 1), jnp.float32)),
        scratch_shapes=[
            pltpu.VMEM((steps, bn, c_in, hw), _MID_DT),
            pltpu.VMEM((c1, 1), jnp.float32),
            pltpu.VMEM((c1, 1), jnp.float32),
            pltpu.VMEM((c2, 1), jnp.float32),
            pltpu.VMEM((c2, 1), jnp.float32),
            pltpu.VMEM((c1, c_in), _DOT_DT),
            pltpu.VMEM((c1, 1), jnp.float32),
            pltpu.VMEM((c2, c1), _DOT_DT),
            pltpu.VMEM((c2, 1), jnp.float32),
        ],
        compiler_params=cp,
    )(x, *params_a)

    params_b = [w1_1, g1_1, b1_1, w2_1, g2_1, b2_1,
                w1_2, g1_2, b1_2, w2_2, g2_2, b2_2]
    out = pl.pallas_call(
        functools.partial(_kernel_b, bn=bn, m_real=m_real),
        grid=(4, steps),
        in_specs=[z0_in_spec, acc_spec(c1), acc_spec(c1)]
                 + [full_spec(a.shape) for a in params_b],
        out_specs=o_spec,
        out_shape=jax.ShapeDtypeStruct((n, c2, h, w), jnp.float32),
        scratch_shapes=[
            pltpu.VMEM((steps, bn, c2, hw), _MID_DT),
            pltpu.VMEM((steps, bn, c2, hw), _MID_DT),
            pltpu.VMEM((c1, 1), jnp.float32),
            pltpu.VMEM((c1, 1), jnp.float32),
            pltpu.VMEM((c2, 1), jnp.float32),
            pltpu.VMEM((c2, 1), jnp.float32),
            pltpu.VMEM((c1, c2), _DOT_DT),
            pltpu.VMEM((c1, 1), jnp.float32),
            pltpu.VMEM((c2, c1), _DOT_DT),
            pltpu.VMEM((c2, 1), jnp.float32),
        ],
        compiler_params=cp,
    )(z0, sum1, ssq1, *params_b)
    return out


# speedup vs baseline: 1.0917x; 1.0136x over previous
import functools

import jax
import jax.numpy as jnp
from jax.experimental import pallas as pl
from jax.experimental.pallas import tpu as pltpu

BN_EPS = 1e-5
LEAKY_SLOPE = 0.2
_DOT_DT = jnp.bfloat16
_MID_DT = jnp.bfloat16


def _lrelu(z):
    return jnp.maximum(z, LEAKY_SLOPE * z)


def _dot(w, a):
    return jnp.dot(w, a.astype(_DOT_DT), preferred_element_type=jnp.float32)


def _gram(a):
    return jax.lax.dot_general(a, a, (((1,), (1,)), ((), ())),
                               preferred_element_type=jnp.float32)


def _fold(s_ref, q_ref, g_ref, b_ref, m_real):
    mean = s_ref[...] / m_real
    var = jnp.maximum(q_ref[...] / m_real - mean * mean, 0.0)
    scale = g_ref[...] * jax.lax.rsqrt(var + BN_EPS)
    shift = b_ref[...] - mean * scale
    return scale, shift


def _fold_from_gram(s_ref, g_ref, w_ref, gam_ref, bet_ref, m_real):
    hi = jax.lax.Precision.HIGHEST
    wt = w_ref[...].astype(_DOT_DT).astype(jnp.float32)
    ch_sum = jnp.dot(wt, s_ref[...], precision=hi,
                     preferred_element_type=jnp.float32)
    p = jnp.dot(wt, g_ref[...], precision=hi,
                preferred_element_type=jnp.float32)
    ch_ssq = jnp.sum(p * wt, axis=1, keepdims=True)
    mean = ch_sum / m_real
    var = jnp.maximum(ch_ssq / m_real - mean * mean, 0.0)
    scale = gam_ref[...] * jax.lax.rsqrt(var + BN_EPS)
    shift = bet_ref[...] - mean * scale
    return scale, shift


def _kernel_a(x_ref, w1_0, g1_0, b1_0, w2_0, g2_0, b2_0, w1_1,
              z0_ref, s_out, g_out,
              buf_a, sg_s, sg_g, sum2, ssq2, w1f, t1s, w2f, t2s,
              *, bn, steps, m_real):
    p = pl.program_id(0)
    j = pl.program_id(1)

    def acc(y, s_ref, q_ref):
        s_ref[...] += jnp.sum(y, axis=1, keepdims=True)
        q_ref[...] += jnp.sum(y * y, axis=1, keepdims=True)

    def acc_sg(ab):
        sg_s[...] += jnp.sum(ab.astype(jnp.float32), axis=1, keepdims=True)
        sg_g[...] += _gram(ab)

    @pl.when(p == 0)
    def _():
        @pl.when(j == 0)
        def _():
            sg_s[...] = jnp.zeros_like(sg_s)
            sg_g[...] = jnp.zeros_like(sg_g)

        ch = x_ref.shape[1]
        m = x_ref.shape[2] * x_ref.shape[3]
        for i in range(bn):
            xb = x_ref[i].reshape(ch, m).astype(_MID_DT)
            buf_a[j, i] = xb
            acc_sg(xb)

    @pl.when(p == 1)
    def _():
        @pl.when(j == 0)
        def _():
            s1, t1v = _fold_from_gram(sg_s, sg_g, w1_0, g1_0, b1_0, m_real)
            w1f[...] = (w1_0[...] * s1).astype(_DOT_DT)
            t1s[...] = t1v
            sum2[...] = jnp.zeros_like(sum2)
            ssq2[...] = jnp.zeros_like(ssq2)

        w1fv = w1f[...]
        w2b = w2_0[...].astype(_DOT_DT)
        for i in range(bn):
            z1 = _lrelu(_dot(w1fv, buf_a[j, i]) + t1s[...])
            acc(_dot(w2b, z1), sum2, ssq2)

    @pl.when(p == 2)
    def _():
        @pl.when(j == 0)
        def _():
            s2, t2v = _fold(sum2, ssq2, g2_0, b2_0, m_real)
            w2f[...] = (w2_0[...] * s2).astype(_DOT_DT)
            t2s[...] = t2v
            sg_s[...] = jnp.zeros_like(sg_s)
            sg_g[...] = jnp.zeros_like(sg_g)

        w1fv = w1f[...]
        w2fv = w2f[...]
        for i in range(bn):
            z1 = _lrelu(_dot(w1fv, buf_a[j, i]) + t1s[...])
            z2b = _lrelu(_dot(w2fv, z1) + t2s[...]).astype(_MID_DT)
            z0_ref[i] = z2b
            acc_sg(z2b)

        @pl.when(j == steps - 1)
        def _():
            s_out[...] = sg_s[...]
            g_out[...] = sg_g[...]


def _kernel_b(z0_ref, s_in, g_in,
              w1_1, g1_1, b1_1, w2_1, g2_1, b2_1,
              w1_2, g1_2, b1_2, w2_2, g2_2, b2_2,
              o_ref,
              buf_a, buf_b, sg_s, sg_g, sum2, ssq2, w1f, t1s, w2f, t2s,
              *, bn, m_real):
    p = pl.program_id(0)
    j = pl.program_id(1)

    def acc(y, s_ref, q_ref):
        s_ref[...] += jnp.sum(y, axis=1, keepdims=True)
        q_ref[...] += jnp.sum(y * y, axis=1, keepdims=True)

    def acc_sg(ab):
        sg_s[...] += jnp.sum(ab.astype(jnp.float32), axis=1, keepdims=True)
        sg_g[...] += _gram(ab)

    @pl.when(p == 0)
    def _():
        @pl.when(j == 0)
        def _():
            s1, t1v = _fold_from_gram(s_in, g_in, w1_1, g1_1, b1_1, m_real)
            w1f[...] = (w1_1[...] * s1).astype(_DOT_DT)
            t1s[...] = t1v
            sum2[...] = jnp.zeros_like(sum2)
            ssq2[...] = jnp.zeros_like(ssq2)

        w1fv = w1f[...]
        w2b = w2_1[...].astype(_DOT_DT)
        for i in range(bn):
            a = z0_ref[i]
            buf_a[j, i] = a
            z1 = _lrelu(_dot(w1fv, a) + t1s[...])
            acc(_dot(w2b, z1), sum2, ssq2)

    @pl.when(p == 1)
    def _():
        @pl.when(j == 0)
        def _():
            s2, t2v = _fold(sum2, ssq2, g2_1, b2_1, m_real)
            w2f[...] = (w2_1[...] * s2).astype(_DOT_DT)
            t2s[...] = t2v
            sg_s[...] = jnp.zeros_like(sg_s)
            sg_g[...] = jnp.zeros_like(sg_g)

        w1fv = w1f[...]
        w2fv = w2f[...]
        for i in range(bn):
            z1 = _lrelu(_dot(w1fv, buf_a[j, i]) + t1s[...])
            z2b = _lrelu(_dot(w2fv, z1) + t2s[...]).astype(_MID_DT)
            buf_b[j, i] = z2b
            acc_sg(z2b)

    @pl.when(p == 2)
    def _():
        @pl.when(j == 0)
        def _():
            s1, t1v = _fold_from_gram(sg_s, sg_g, w1_2, g1_2, b1_2, m_real)
            w1f[...] = (w1_2[...] * s1).astype(_DOT_DT)
            t1s[...] = t1v
            sum2[...] = jnp.zeros_like(sum2)
            ssq2[...] = jnp.zeros_like(ssq2)

        w1fv = w1f[...]
        w2b = w2_2[...].astype(_DOT_DT)
        for i in range(bn):
            z1 = _lrelu(_dot(w1fv, buf_b[j, i]) + t1s[...])
            acc(_dot(w2b, z1), sum2, ssq2)

    @pl.when(p == 3)
    def _():
        @pl.when(j == 0)
        def _():
            s2, t2v = _fold(sum2, ssq2, g2_2, b2_2, m_real)
            w2f[...] = (w2_2[...] * s2).astype(_DOT_DT)
            t2s[...] = t2v

        w1fv = w1f[...]
        w2fv = w2f[...]
        ch, hh, ww = o_ref.shape[1], o_ref.shape[2], o_ref.shape[3]
        for i in range(bn):
            z1 = _lrelu(_dot(w1fv, buf_b[j, i]) + t1s[...])
            z2 = _lrelu(_dot(w2fv, z1) + t2s[...])
            o_ref[i] = z2.reshape(ch, hh, ww)


def kernel(x,
           w1_0, g1_0, b1_0, w2_0, g2_0, b2_0,
           w1_1, g1_1, b1_1, w2_1, g2_1, b2_1,
           w1_2, g1_2, b1_2, w2_2, g2_2, b2_2):
    n, c_in, h, w = x.shape
    hw = h * w
    m_real = n * hw
    c1 = w1_0.shape[0]
    c2 = w2_0.shape[0]

    bn = 4
    steps = -(-n // bn)

    cp = pltpu.CompilerParams(
        dimension_semantics=("arbitrary", "arbitrary"),
        vmem_limit_bytes=56 * 1024 * 1024)

    def full_spec(shape):
        nd = len(shape)
        return pl.BlockSpec(tuple(shape), lambda p, j: (0,) * nd)

    def acc_spec(ch):
        return pl.BlockSpec((ch, 1), lambda p, j: (0, 0))

    last = steps - 1
    x_spec = pl.BlockSpec(
        (bn, c_in, h, w),
        lambda p, j: (jnp.where(p == 0, j, last), 0, 0, 0))
    z0_out_spec = pl.BlockSpec(
        (bn, c2, hw), lambda p, j: (jnp.where(p == 2, j, 0), 0, 0))
    z0_in_spec = pl.BlockSpec(
        (bn, c2, hw), lambda p, j: (jnp.where(p == 0, j, last), 0, 0))
    o_spec = pl.BlockSpec(
        (bn, c2, h, w),
        lambda p, j: (jnp.where(p == 3, j, 0), 0, 0, 0))

    def gram_spec():
        return pl.BlockSpec((c2, c2), lambda p, j: (0, 0))

    params_a = [w1_0, g1_0, b1_0, w2_0, g2_0, b2_0, w1_1]
    z0, s01, g01 = pl.pallas_call(
        functools.partial(_kernel_a, bn=bn, steps=steps, m_real=m_real),
        grid=(3, steps),
        in_specs=[x_spec] + [full_spec(a.shape) for a in params_a],
        out_specs=(z0_out_spec, acc_spec(c2), gram_spec()),
        out_shape=(jax.ShapeDtypeStruct((n, c2, hw), _MID_DT),
                   jax.ShapeDtypeStruct((c2, 1), jnp.float32),
                   jax.ShapeDtypeStruct((c2, c2), jnp.float32)),
        scratch_shapes=[
            pltpu.VMEM((steps, bn, c_in, hw), _MID_DT),
            pltpu.VMEM((c2, 1), jnp.float32),
            pltpu.VMEM((c2, c2), jnp.float32),
            pltpu.VMEM((c2, 1), jnp.float32),
            pltpu.VMEM((c2, 1), jnp.float32),
            pltpu.VMEM((c1, c_in), _DOT_DT),
            pltpu.VMEM((c1, 1), jnp.float32),
            pltpu.VMEM((c2, c1), _DOT_DT),
            pltpu.VMEM((c2, 1), jnp.float32),
        ],
        compiler_params=cp,
    )(x, *params_a)

    params_b = [w1_1, g1_1, b1_1, w2_1, g2_1, b2_1,
                w1_2, g1_2, b1_2, w2_2, g2_2, b2_2]
    out = pl.pallas_call(
        functools.partial(_kernel_b, bn=bn, m_real=m_real),
        grid=(4, steps),
        in_specs=[z0_in_spec, acc_spec(c2), gram_spec()]
                 + [full_spec(a.shape) for a in params_b],
        out_specs=o_spec,
        out_shape=jax.ShapeDtypeStruct((n, c2, h, w), jnp.float32),
        scratch_shapes=[
            pltpu.VMEM((steps, bn, c2, hw), _MID_DT),
            pltpu.VMEM((steps, bn, c2, hw), _MID_DT),
            pltpu.VMEM((c2, 1), jnp.float32),
            pltpu.VMEM((c2, c2), jnp.float32),
            pltpu.VMEM((c2, 1), jnp.float32),
            pltpu.VMEM((c2, 1), jnp.float32),
            pltpu.VMEM((c1, c2), _DOT_DT),
            pltpu.VMEM((c1, 1), jnp.float32),
            pltpu.VMEM((c2, c1), _DOT_DT),
            pltpu.VMEM((c2, 1), jnp.float32),
        ],
        compiler_params=cp,
    )(z0, s01, g01, *params_b)
    return out
